# Initial kernel scaffold; baseline (speedup 1.0000x reference)
import jax
import jax.numpy as jnp
from jax import lax
from jax.experimental import pallas as pl
from jax.experimental.pallas import tpu as pltpu
from jax.experimental.pallas import tpu_sc as plsc

_GRID = 128
_N = 2097152
_NC = 2
_NS = 16
_NW = _NC * _NS
_P = _N // _NW
_C = 2048
_NCHUNK = _P // _C
_L = 16


def _tec_body(xT_hbm, grid_hbm, out_hbm,
              cx, cy, cz, wx, wy, wz, acc,
              idx0, idx1, idx2, idx3, idx4, idx5, idx6, idx7,
              v0, v1, v2, v3, v4, v5, v6, v7,
              sem):
    idxs = (idx0, idx1, idx2, idx3, idx4, idx5, idx6, idx7)
    vals = (v0, v1, v2, v3, v4, v5, v6, v7)
    wid = lax.axis_index("s") * _NC + lax.axis_index("c")
    base = wid * _P

    def chunk_body(c, carry):
        off = base + c * _C
        pltpu.sync_copy(xT_hbm.at[0, pl.ds(off, _C)], cx)
        pltpu.sync_copy(xT_hbm.at[1, pl.ds(off, _C)], cy)
        pltpu.sync_copy(xT_hbm.at[2, pl.ds(off, _C)], cz)

        def pass1(i, carry1):
            s = pl.ds(i * _L, _L)
            fx = jnp.clip(cx[s] * 64.0 + 63.5, 0.0, 127.0)
            fy = jnp.clip(cy[s] * 64.0 + 63.5, 0.0, 127.0)
            fz = jnp.clip(cz[s] * 64.0 + 63.5, 0.0, 127.0)
            x0 = fx.astype(jnp.int32)
            y0 = fy.astype(jnp.int32)
            z0 = fz.astype(jnp.int32)
            wx[s] = fx - x0.astype(jnp.float32)
            wy[s] = fy - y0.astype(jnp.float32)
            wz[s] = fz - z0.astype(jnp.float32)
            dx = jnp.minimum(x0 + 1, 127) - x0
            dy = (jnp.minimum(y0 + 1, 127) - y0) * 128
            dz = (jnp.minimum(z0 + 1, 127) - z0) * 16384
            f000 = z0 * 16384 + y0 * 128 + x0
            idx0[s] = f000
            idx1[s] = f000 + dx
            idx2[s] = f000 + dy
            idx3[s] = f000 + dy + dx
            idx4[s] = f000 + dz
            idx5[s] = f000 + dz + dx
            idx6[s] = f000 + dz + dy
            idx7[s] = f000 + dz + dy + dx
            return carry1

        lax.fori_loop(0, _C // _L, pass1, 0)

        cps = [pltpu.async_copy(grid_hbm.at[ix], v, sem)
               for ix, v in zip(idxs, vals)]
        for cp in cps:
            cp.wait()

        def pass3(i, carry3):
            s = pl.ds(i * _L, _L)
            fx = wx[s]
            fy = wy[s]
            fz = wz[s]
            gx = 1.0 - fx
            gy = 1.0 - fy
            gz = 1.0 - fz
            acc[s] = (gz * (gy * (v0[s] * gx + v1[s] * fx)
                            + fy * (v2[s] * gx + v3[s] * fx))
                      + fz * (gy * (v4[s] * gx + v5[s] * fx)
                              + fy * (v6[s] * gx + v7[s] * fx)))
            return carry3

        lax.fori_loop(0, _C // _L, pass3, 0)
        pltpu.sync_copy(acc, out_hbm.at[pl.ds(off, _C)])
        return carry

    lax.fori_loop(0, _NCHUNK, chunk_body, 0)


def kernel(x, base_grid):
    xT = x.T
    grid_flat = base_grid.reshape(-1)
    f = pl.kernel(
        _tec_body,
        out_type=jax.ShapeDtypeStruct((_N,), jnp.float32),
        mesh=plsc.VectorSubcoreMesh(core_axis_name="c", subcore_axis_name="s"),
        scratch_types=(
            [pltpu.VMEM((_C,), jnp.float32) for _ in range(7)]
            + [pltpu.VMEM((_C,), jnp.int32) for _ in range(8)]
            + [pltpu.VMEM((_C,), jnp.float32) for _ in range(8)]
            + [pltpu.SemaphoreType.DMA]
        ),
    )
    return f(xT, grid_flat)


# speedup vs baseline: 1.3239x; 1.3239x over previous
import jax
import jax.numpy as jnp
from jax import lax
from jax.experimental import pallas as pl
from jax.experimental.pallas import tpu as pltpu
from jax.experimental.pallas import tpu_sc as plsc

_GRID = 128
_N = 2097152
_NC = 2
_NS = 16
_NW = _NC * _NS
_P = _N // _NW
_C = 2048
_NCHUNK = _P // _C
_L = 16


def _tec_body(cx_hbm, cy_hbm, cz_hbm, grid_hbm, out_hbm,
              cx, cy, cz, wx, wy, wz, acc,
              idx0, idx1, idx2, idx3, idx4, idx5, idx6, idx7,
              v0, v1, v2, v3, v4, v5, v6, v7,
              sem):
    idxs = (idx0, idx1, idx2, idx3, idx4, idx5, idx6, idx7)
    vals = (v0, v1, v2, v3, v4, v5, v6, v7)
    wid = lax.axis_index("s") * _NC + lax.axis_index("c")
    base = wid * _P

    def chunk_body(c, carry):
        off = base + c * _C
        pltpu.sync_copy(cx_hbm.at[pl.ds(off, _C)], cx)
        pltpu.sync_copy(cy_hbm.at[pl.ds(off, _C)], cy)
        pltpu.sync_copy(cz_hbm.at[pl.ds(off, _C)], cz)

        def pass1(i, carry1):
            s = pl.ds(i * _L, _L)
            fx = jnp.clip(cx[s] * 64.0 + 63.5, 0.0, 127.0)
            fy = jnp.clip(cy[s] * 64.0 + 63.5, 0.0, 127.0)
            fz = jnp.clip(cz[s] * 64.0 + 63.5, 0.0, 127.0)
            x0 = fx.astype(jnp.int32)
            y0 = fy.astype(jnp.int32)
            z0 = fz.astype(jnp.int32)
            wx[s] = fx - x0.astype(jnp.float32)
            wy[s] = fy - y0.astype(jnp.float32)
            wz[s] = fz - z0.astype(jnp.float32)
            dx = jnp.minimum(x0 + 1, 127) - x0
            dy = (jnp.minimum(y0 + 1, 127) - y0) * 128
            dz = (jnp.minimum(z0 + 1, 127) - z0) * 16384
            f000 = z0 * 16384 + y0 * 128 + x0
            idx0[s] = f000
            idx1[s] = f000 + dx
            idx2[s] = f000 + dy
            idx3[s] = f000 + dy + dx
            idx4[s] = f000 + dz
            idx5[s] = f000 + dz + dx
            idx6[s] = f000 + dz + dy
            idx7[s] = f000 + dz + dy + dx
            return carry1

        lax.fori_loop(0, _C // _L, pass1, 0)

        cps = [pltpu.async_copy(grid_hbm.at[ix], v, sem)
               for ix, v in zip(idxs, vals)]
        for cp in cps:
            cp.wait()

        def pass3(i, carry3):
            s = pl.ds(i * _L, _L)
            fx = wx[s]
            fy = wy[s]
            fz = wz[s]
            gx = 1.0 - fx
            gy = 1.0 - fy
            gz = 1.0 - fz
            acc[s] = (gz * (gy * (v0[s] * gx + v1[s] * fx)
                            + fy * (v2[s] * gx + v3[s] * fx))
                      + fz * (gy * (v4[s] * gx + v5[s] * fx)
                              + fy * (v6[s] * gx + v7[s] * fx)))
            return carry3

        lax.fori_loop(0, _C // _L, pass3, 0)
        pltpu.sync_copy(acc, out_hbm.at[pl.ds(off, _C)])
        return carry

    lax.fori_loop(0, _NCHUNK, chunk_body, 0)


def kernel(x, base_grid):
    xT = x.T
    grid_flat = base_grid.reshape(-1)
    f = pl.kernel(
        _tec_body,
        out_type=jax.ShapeDtypeStruct((_N,), jnp.float32),
        mesh=plsc.VectorSubcoreMesh(core_axis_name="c", subcore_axis_name="s"),
        scratch_types=(
            [pltpu.VMEM((_C,), jnp.float32) for _ in range(7)]
            + [pltpu.VMEM((_C,), jnp.int32) for _ in range(8)]
            + [pltpu.VMEM((_C,), jnp.float32) for _ in range(8)]
            + [pltpu.SemaphoreType.DMA]
        ),
    )
    return f(xT[0], xT[1], xT[2], grid_flat)


# speedup vs baseline: 2.7873x; 2.1053x over previous
import jax
import jax.numpy as jnp
from jax import lax
from jax.experimental import pallas as pl
from jax.experimental.pallas import tpu as pltpu
from jax.experimental.pallas import tpu_sc as plsc

_GRID = 128
_N = 2097152
_NC = 2
_NS = 16
_NW = _NC * _NS
_P = _N // _NW
_C = 2048
_NCHUNK = _P // _C
_NPAIR = _NCHUNK // 2
_L = 16


def _pack_body(g_ref, o_ref):
    g = g_ref[...]
    col = lax.broadcasted_iota(jnp.int32, g.shape, 1)
    nxt = jnp.where(col < _GRID - 1, pltpu.roll(g, _GRID - 1, 1), g)
    ua = lax.bitcast_convert_type(g, jnp.uint32)
    ub = lax.bitcast_convert_type(nxt, jnp.uint32)
    ua = ua + jnp.uint32(0x7FFF) + ((ua >> 16) & jnp.uint32(1))
    ub = ub + jnp.uint32(0x7FFF) + ((ub >> 16) & jnp.uint32(1))
    w = (ua & jnp.uint32(0xFFFF0000)) | (ub >> 16)
    o_ref[...] = lax.bitcast_convert_type(w, jnp.int32)


def _tec_body(cx_hbm, cy_hbm, cz_hbm, tab_hbm, out_hbm, *scr):
    A = scr[0:15]
    B = scr[15:30]
    csemA, csemB, gsemA, gsemB, osemA, osemB = scr[30:36]
    wid = lax.axis_index("s") * _NC + lax.axis_index("c")
    base = wid * _P

    coords_hbm = (cx_hbm, cy_hbm, cz_hbm)

    def fire_coords(bufs, sem, c):
        off = base + c * _C
        for h, d in zip(coords_hbm, bufs[0:3]):
            pltpu.async_copy(h.at[pl.ds(off, _C)], d, sem)

    def drain_coords(bufs, sem):
        for h, d in zip(coords_hbm, bufs[0:3]):
            pltpu.make_async_copy(h.at[pl.ds(0, _C)], d, sem).wait()

    def pass1(bufs):
        cx, cy, cz, wx, wy, wz = bufs[0:6]
        idx0, idx1, idx2, idx3 = bufs[7:11]

        def body(i, carry):
            s = pl.ds(i * _L, _L)
            fx = jnp.clip(cx[s] * 64.0 + 63.5, 0.0, 127.0)
            fy = jnp.clip(cy[s] * 64.0 + 63.5, 0.0, 127.0)
            fz = jnp.clip(cz[s] * 64.0 + 63.5, 0.0, 127.0)
            x0 = fx.astype(jnp.int32)
            y0 = fy.astype(jnp.int32)
            z0 = fz.astype(jnp.int32)
            wx[s] = fx - x0.astype(jnp.float32)
            wy[s] = fy - y0.astype(jnp.float32)
            wz[s] = fz - z0.astype(jnp.float32)
            dy = (jnp.minimum(y0 + 1, 127) - y0) * 128
            dz = (jnp.minimum(z0 + 1, 127) - z0) * 16384
            f00 = z0 * 16384 + y0 * 128 + x0
            idx0[s] = f00
            idx1[s] = f00 + dy
            idx2[s] = f00 + dz
            idx3[s] = f00 + dz + dy
            return carry

        lax.fori_loop(0, _C // _L, body, 0)

    def fire_gathers(bufs, sem):
        for ix, v in zip(bufs[7:11], bufs[11:15]):
            pltpu.async_copy(tab_hbm.at[ix], v, sem)

    def drain_gathers(bufs, sem):
        for ix, v in zip(bufs[7:11], bufs[11:15]):
            pltpu.make_async_copy(tab_hbm.at[ix], v, sem).wait()

    def pass3(bufs):
        wx, wy, wz, acc = bufs[3:7]
        v0, v1, v2, v3 = bufs[11:15]

        def body(i, carry):
            s = pl.ds(i * _L, _L)
            fx = wx[s]
            fy = wy[s]
            fz = wz[s]
            hi = jnp.int32(-65536)

            def lerp_x(w):
                a = lax.bitcast_convert_type(w & hi, jnp.float32)
                b = lax.bitcast_convert_type(w << 16, jnp.float32)
                return a + fx * (b - a)

            l00 = lerp_x(v0[s])
            l01 = lerp_x(v1[s])
            l10 = lerp_x(v2[s])
            l11 = lerp_x(v3[s])
            m0 = l00 + fy * (l01 - l00)
            m1 = l10 + fy * (l11 - l10)
            acc[s] = m0 + fz * (m1 - m0)
            return carry

        lax.fori_loop(0, _C // _L, body, 0)

    def fire_out(bufs, sem, c):
        off = base + c * _C
        pltpu.async_copy(bufs[6], out_hbm.at[pl.ds(off, _C)], sem)

    def drain_out(bufs, sem):
        pltpu.make_async_copy(bufs[6], out_hbm.at[pl.ds(0, _C)], sem).wait()

    fire_coords(A, csemA, 0)

    def pair_body(kk, carry):
        c0 = 2 * kk
        fire_coords(B, csemB, c0 + 1)
        drain_coords(A, csemA)
        pass1(A)
        fire_gathers(A, gsemA)

        @pl.when(kk > 1)
        def _older_b_out():
            drain_out(B, osemB)

        @pl.when(kk > 0)
        def _older_b():
            drain_gathers(B, gsemB)
            pass3(B)
            fire_out(B, osemB, c0 - 1)

        @pl.when(kk < _NPAIR - 1)
        def _prefetch_a():
            fire_coords(A, csemA, c0 + 2)

        drain_coords(B, csemB)
        pass1(B)
        fire_gathers(B, gsemB)

        drain_gathers(A, gsemA)

        @pl.when(kk > 0)
        def _older_a_out():
            drain_out(A, osemA)

        pass3(A)
        fire_out(A, osemA, c0)
        return carry

    lax.fori_loop(0, _NPAIR, pair_body, 0)

    drain_gathers(B, gsemB)
    drain_out(B, osemB)
    pass3(B)
    pltpu.sync_copy(B[6], out_hbm.at[pl.ds(base + (_NCHUNK - 1) * _C, _C)])
    drain_out(A, osemA)


def kernel(x, base_grid):
    xT = x.T
    g2d = base_grid.reshape(_GRID * _GRID, _GRID)
    packed = pl.pallas_call(
        _pack_body,
        grid=(8,),
        in_specs=[pl.BlockSpec((_GRID * _GRID // 8, _GRID), lambda i: (i, 0))],
        out_specs=pl.BlockSpec((_GRID * _GRID // 8, _GRID), lambda i: (i, 0)),
        out_shape=jax.ShapeDtypeStruct((_GRID * _GRID, _GRID), jnp.int32),
    )(g2d)
    table = packed.reshape(-1)
    bufset = ([pltpu.VMEM((_C,), jnp.float32) for _ in range(7)]
              + [pltpu.VMEM((_C,), jnp.int32) for _ in range(8)])
    f = pl.kernel(
        _tec_body,
        out_type=jax.ShapeDtypeStruct((_N,), jnp.float32),
        mesh=plsc.VectorSubcoreMesh(core_axis_name="c", subcore_axis_name="s"),
        scratch_types=(bufset + bufset
                       + [pltpu.SemaphoreType.DMA for _ in range(6)]),
    )
    return f(xT[0], xT[1], xT[2], table)
